# baseline (speedup 1.0000x reference)
import dataclasses
import functools

import jax
import jax.numpy as jnp
from jax import lax
from jax.experimental import pallas as pl
from jax.experimental.pallas import tpu as pltpu
from jax.experimental.pallas import tpu_sc as plsc

H = 480
W = 640
RY = 8
RX = 8
GH = H + 2 * RY
GW = W + 2 * RX
GHW = GH * GW
ACC = 2 * GHW
NTILES = 16
BATCHES_PER_CORE = 4
CHUNK = 800
NVEC = CHUNK // 16
ENTRIES = 4 * CHUNK
SLICE128 = 40576
TAIL128 = ACC - NTILES * SLICE128
ZCHUNK = 8192
ZTAIL = SLICE128 - 4 * ZCHUNK


def _splat(ts_flat, loc_flat, flow_flat, pol_flat, batches, n):
    nchunks = n // CHUNK
    mesh = plsc.VectorSubcoreMesh(core_axis_name="c", subcore_axis_name="s")
    cp = pltpu.CompilerParams()
    if "needs_layout_passes" in pltpu.CompilerParams.__dataclass_fields__:
        cp = dataclasses.replace(cp, needs_layout_passes=False)

    @functools.partial(
        pl.kernel,
        compiler_params=cp,
        out_type=jax.ShapeDtypeStruct((batches * ACC,), jnp.float32),
        mesh=mesh,
        scratch_types=[
            pltpu.VMEM((CHUNK,), jnp.float32),
            pltpu.VMEM((2 * CHUNK,), jnp.float32),
            pltpu.VMEM((2 * CHUNK,), jnp.float32),
            pltpu.VMEM((2 * CHUNK,), jnp.float32),
            pltpu.VMEM((ENTRIES,), jnp.int32),
            pltpu.VMEM((ENTRIES,), jnp.float32),
            pltpu.VMEM((ZCHUNK,), jnp.float32),
            pltpu.VMEM_SHARED((ACC,), jnp.float32),
            pltpu.SemaphoreType.DMA,
        ],
    )
    def k(ts_hbm, loc_hbm, flow_hbm, pol_hbm, out_hbm,
          ts_v, loc_v, flow_v, pol_v, idx_v, val_v, zero_v, acc_sh, sem):
        c = lax.axis_index("c")
        s = lax.axis_index("s")
        nc = jnp.where(s < nchunks - (nchunks // NTILES) * NTILES,
                       nchunks // NTILES + 1, nchunks // NTILES)
        lane = jax.lax.iota(jnp.int32, 16)
        lane2 = lane * 2

        @pl.loop(0, ZCHUNK // 16)
        def _(i):
            zero_v[pl.ds(i * 16, 16)] = jnp.zeros((16,), jnp.float32)

        @pl.loop(0, BATCHES_PER_CORE)
        def _(bi):
            b = c * BATCHES_PER_CORE + bi

            @pl.loop(0, 4)
            def _(zi):
                pltpu.sync_copy(
                    zero_v, acc_sh.at[pl.ds(s * SLICE128 + zi * ZCHUNK, ZCHUNK)])
            pltpu.sync_copy(
                zero_v.at[pl.ds(0, ZTAIL)],
                acc_sh.at[pl.ds(s * SLICE128 + 4 * ZCHUNK, ZTAIL)])

            @pl.when(s == 0)
            def _():
                pltpu.sync_copy(zero_v.at[pl.ds(0, TAIL128)],
                                acc_sh.at[pl.ds(NTILES * SLICE128, TAIL128)])
            plsc.subcore_barrier()

            @pl.loop(0, (nchunks + NTILES - 1) // NTILES)
            def _(ki):
                @pl.when(ki < nc)
                def _():
                    off = (s + ki * NTILES) * CHUNK
                    d0 = pltpu.async_copy(
                        ts_hbm.at[pl.ds(b * n + off, CHUNK)], ts_v, sem)
                    d1 = pltpu.async_copy(
                        loc_hbm.at[pl.ds(2 * (b * n + off), 2 * CHUNK)], loc_v, sem)
                    d2 = pltpu.async_copy(
                        flow_hbm.at[pl.ds(2 * (b * n + off), 2 * CHUNK)], flow_v, sem)
                    d3 = pltpu.async_copy(
                        pol_hbm.at[pl.ds(2 * (b * n + off), 2 * CHUNK)], pol_v, sem)
                    d0.wait()
                    d1.wait()
                    d2.wait()
                    d3.wait()

                    @pl.loop(0, NVEC)
                    def _(vi):
                        ts = ts_v[pl.ds(vi * 16, 16)]
                        iy = lane2 + vi * 32
                        ix = iy + 1
                        ly = plsc.load_gather(loc_v, [iy])
                        lx = plsc.load_gather(loc_v, [ix])
                        fy = plsc.load_gather(flow_v, [iy])
                        fx = plsc.load_gather(flow_v, [ix])
                        po = plsc.load_gather(pol_v, [iy])
                        t = 1.0 - ts
                        wy = ly + t * fy
                        wx = lx + t * fx
                        yi = (wy + 512.0).astype(jnp.int32)
                        dy = wy - (yi.astype(jnp.float32) - 512.0)
                        xi = (wx + 512.0).astype(jnp.int32)
                        dx = wx - (xi.astype(jnp.float32) - 512.0)
                        gy = jnp.minimum(jnp.maximum(yi - (512 - RY), 0), GH - 2)
                        gx = jnp.minimum(jnp.maximum(xi - (512 - RX), 0), GW - 2)
                        pz = po.astype(jnp.int32)
                        base = (1 - pz) * GHW + gy * GW + gx
                        uy = 1.0 - dy
                        ux = 1.0 - dx
                        o = vi * 64
                        idx_v[pl.ds(o, 16)] = base
                        val_v[pl.ds(o, 16)] = uy * ux
                        idx_v[pl.ds(o + 16, 16)] = base + 1
                        val_v[pl.ds(o + 16, 16)] = uy * dx
                        idx_v[pl.ds(o + 32, 16)] = base + GW
                        val_v[pl.ds(o + 32, 16)] = dy * ux
                        idx_v[pl.ds(o + 48, 16)] = base + GW + 1
                        val_v[pl.ds(o + 48, 16)] = dy * dx

                    pltpu.sync_copy(val_v, acc_sh.at[idx_v], add=True)

            plsc.subcore_barrier()
            out_base = b * ACC
            pltpu.sync_copy(
                acc_sh.at[pl.ds(s * SLICE128, SLICE128)],
                out_hbm.at[pl.ds(out_base + s * SLICE128, SLICE128)])

            @pl.when(s == 0)
            def _():
                pltpu.sync_copy(
                    acc_sh.at[pl.ds(NTILES * SLICE128, TAIL128)],
                    out_hbm.at[pl.ds(out_base + NTILES * SLICE128, TAIL128)])

    return k(ts_flat, loc_flat, flow_flat, pol_flat)


def kernel(event_ts, event_loc, event_flow, pol_mask):
    batches, n, _ = event_ts.shape
    out = _splat(event_ts.reshape(-1), event_loc.reshape(-1),
                 event_flow.reshape(-1), pol_mask.reshape(-1), batches, n)
    out = out.reshape(batches, 2, GH, GW)
    return out[:, :, RY:RY + H, RX:RX + W]


# speedup vs baseline: 11.7982x; 11.7982x over previous
import dataclasses
import functools

import jax
import jax.numpy as jnp
from jax import lax
from jax.experimental import pallas as pl
from jax.experimental.pallas import tpu as pltpu
from jax.experimental.pallas import tpu_sc as plsc

H = 480
W = 640
RY = 8
GH = H + 2 * RY
PLANE = GH * W
ACC = 2 * PLANE
INT_OFF = RY * W
OUT_B = 2 * H * W
NTILES = 16
BPC = 4
CHUNK = 1792
NVEC = CHUNK // 16
ENTRIES = 4 * CHUNK
EV = 6 * CHUNK
ZS = ACC // NTILES
WS = OUT_B // NTILES


def _splat(pk, batches, nch):
    chunks_per_tile = nch // NTILES
    mesh = plsc.VectorSubcoreMesh(core_axis_name="c", subcore_axis_name="s")
    cp = pltpu.CompilerParams()
    if "needs_layout_passes" in pltpu.CompilerParams.__dataclass_fields__:
        cp = dataclasses.replace(cp, needs_layout_passes=False)

    @functools.partial(
        pl.kernel,
        compiler_params=cp,
        out_type=jax.ShapeDtypeStruct((batches * OUT_B,), jnp.float32),
        mesh=mesh,
        scratch_types=[
            pltpu.VMEM((EV,), jnp.float32),
            pltpu.VMEM((ENTRIES,), jnp.int32),
            pltpu.VMEM((ENTRIES,), jnp.float32),
            pltpu.VMEM_SHARED((ACC,), jnp.float32),
            pltpu.SemaphoreType.DMA,
        ],
    )
    def k(pk_hbm, out_hbm, ev_v, idx_v, val_v, acc_sh, sem):
        c = lax.axis_index("c")
        s = lax.axis_index("s")

        @pl.loop(0, BPC)
        def _(bi):
            b = c * BPC + bi

            @pl.loop(0, ENTRIES // 16)
            def _(i):
                val_v[pl.ds(i * 16, 16)] = jnp.zeros((16,), jnp.float32)

            @pl.loop(0, ZS // ENTRIES)
            def _(zi):
                pltpu.sync_copy(
                    val_v, acc_sh.at[pl.ds(s * ZS + zi * ENTRIES, ENTRIES)])
            pltpu.sync_copy(
                val_v.at[pl.ds(0, ZS - (ZS // ENTRIES) * ENTRIES)],
                acc_sh.at[pl.ds(s * ZS + (ZS // ENTRIES) * ENTRIES,
                                ZS - (ZS // ENTRIES) * ENTRIES)])
            plsc.subcore_barrier()

            @pl.loop(0, chunks_per_tile)
            def _(ki):
                off = (b * nch + s + ki * NTILES) * EV
                pltpu.async_copy(pk_hbm.at[pl.ds(off, EV)], ev_v, sem).wait()

                @pl.loop(0, NVEC)
                def _(vi):
                    o16 = vi * 16
                    ts = ev_v[pl.ds(o16, 16)]
                    ly = ev_v[pl.ds(CHUNK + o16, 16)]
                    lx = ev_v[pl.ds(2 * CHUNK + o16, 16)]
                    fy = ev_v[pl.ds(3 * CHUNK + o16, 16)]
                    fx = ev_v[pl.ds(4 * CHUNK + o16, 16)]
                    po = ev_v[pl.ds(5 * CHUNK + o16, 16)]
                    t = 1.0 - ts
                    wy = ly + t * fy
                    wx = lx + t * fx
                    yi = (wy + 512.0).astype(jnp.int32)
                    dy = wy - (yi.astype(jnp.float32) - 512.0)
                    xi = (wx + 512.0).astype(jnp.int32)
                    dx = wx - (xi.astype(jnp.float32) - 512.0)
                    yg = jnp.minimum(jnp.maximum(yi - (512 - RY), 0), GH - 2)
                    x0 = xi - 512
                    m0 = (x0 >= 0) & (x0 <= W - 1)
                    m1 = (x0 >= -1) & (x0 <= W - 2)
                    cx0 = jnp.minimum(jnp.maximum(x0, 0), W - 1)
                    cx1 = jnp.minimum(jnp.maximum(x0 + 1, 0), W - 1)
                    pz = po.astype(jnp.int32)
                    rb = (1 - pz) * PLANE + yg * W
                    uy = 1.0 - dy
                    ux = 1.0 - dx
                    zv = jnp.zeros((16,), jnp.float32)
                    o = vi * 64
                    idx_v[pl.ds(o, 16)] = rb + cx0
                    val_v[pl.ds(o, 16)] = jnp.where(m0, uy * ux, zv)
                    idx_v[pl.ds(o + 16, 16)] = rb + cx1
                    val_v[pl.ds(o + 16, 16)] = jnp.where(m1, uy * dx, zv)
                    idx_v[pl.ds(o + 32, 16)] = rb + W + cx0
                    val_v[pl.ds(o + 32, 16)] = jnp.where(m0, dy * ux, zv)
                    idx_v[pl.ds(o + 48, 16)] = rb + W + cx1
                    val_v[pl.ds(o + 48, 16)] = jnp.where(m1, dy * dx, zv)

                pltpu.sync_copy(val_v, acc_sh.at[idx_v], add=True)

            plsc.subcore_barrier()
            acc_off = (s // 8) * PLANE + INT_OFF + (s % 8) * WS
            pltpu.sync_copy(
                acc_sh.at[pl.ds(acc_off, WS)],
                out_hbm.at[pl.ds(b * OUT_B + s * WS, WS)])
            plsc.subcore_barrier()

    return k(pk)


def kernel(event_ts, event_loc, event_flow, pol_mask):
    B, N, _ = event_ts.shape
    step = CHUNK * NTILES
    n_pad = ((N + step - 1) // step) * step
    nch = n_pad // CHUNK
    ts = event_ts[..., 0]
    po = pol_mask[..., 0]
    pk = jnp.stack([ts, event_loc[..., 0], event_loc[..., 1],
                    event_flow[..., 0], event_flow[..., 1], po], axis=1)
    if n_pad > N:
        padblk = jnp.zeros((B, 6, n_pad - N), jnp.float32).at[:, 2, :].set(-1e6)
        pk = jnp.concatenate([pk, padblk], axis=2)
    pk = pk.reshape(B, 6, nch, CHUNK).transpose(0, 2, 1, 3).reshape(-1)
    out = _splat(pk, B, nch)
    return out.reshape(B, 2, H, W)
